# Initial kernel scaffold; baseline (speedup 1.0000x reference)
import jax
import jax.numpy as jnp
from jax.experimental import pallas as pl


def kernel(indices, embeddings):
    raise NotImplementedError("write your pallas kernel here")


# speedup vs baseline: 1.6846x; 1.6846x over previous
import functools

import jax
import jax.numpy as jnp
from jax import lax
from jax.experimental import pallas as pl
from jax.experimental.pallas import tpu as pltpu
from jax.experimental.pallas import tpu_sc as plsc

_CHUNK = 128

_info = plsc.get_sparse_core_info()
_NC = _info.num_cores
_NS = _info.num_subcores
_NW = _NC * _NS


@functools.partial(jax.jit, static_argnames=("n_chunks_per_w", "dim"))
def _gather(idx2d, table, *, n_chunks_per_w, dim):
    n_idx = idx2d.shape[0] * idx2d.shape[1]
    mesh = plsc.VectorSubcoreMesh(core_axis_name="c", subcore_axis_name="s")

    @functools.partial(
        pl.kernel,
        out_type=jax.ShapeDtypeStruct((n_idx, dim), jnp.float32),
        mesh=mesh,
        scratch_types=[
            pltpu.VMEM((n_chunks_per_w, _CHUNK), jnp.int32),
            pltpu.VMEM((_CHUNK, dim), jnp.float32),
            pltpu.SemaphoreType.DMA,
        ],
        compiler_params=pltpu.CompilerParams(use_tc_tiling_on_sc=False),
    )
    def k(idx_hbm, table_hbm, out_hbm, idx_v, rows_v, sem):
        wid = lax.axis_index("s") * _NC + lax.axis_index("c")
        row0 = wid * n_chunks_per_w
        pltpu.sync_copy(idx_hbm.at[pl.ds(row0, n_chunks_per_w)], idx_v)

        @pl.loop(0, n_chunks_per_w)
        def chunk_body(j):
            pltpu.async_copy(table_hbm.at[idx_v.at[j]], rows_v, sem).wait()
            pltpu.sync_copy(rows_v, out_hbm.at[pl.ds((row0 + j) * _CHUNK, _CHUNK)])

    return k(idx2d, table)


def kernel(indices, embeddings):
    b, s = indices.shape
    dim = embeddings.shape[1]
    n_idx = b * s
    assert n_idx % (_CHUNK * _NW) == 0
    idx2d = indices.reshape(n_idx // _CHUNK, _CHUNK)
    out = _gather(idx2d, embeddings, n_chunks_per_w=n_idx // (_CHUNK * _NW), dim=dim)
    return out.reshape(b, s, dim)


# speedup vs baseline: 1.8768x; 1.1141x over previous
import functools

import jax
import jax.numpy as jnp
from jax import lax
from jax.experimental import pallas as pl
from jax.experimental.pallas import tpu as pltpu
from jax.experimental.pallas import tpu_sc as plsc

_CHUNK = 128
_GPB = 2
_NBUF = 4

_info = plsc.get_sparse_core_info()
_NC = _info.num_cores
_NS = _info.num_subcores
_NW = _NC * _NS


@functools.partial(jax.jit, static_argnames=("n_chunks_per_w", "dim"))
def _gather(idx2d, table, *, n_chunks_per_w, dim):
    n_idx = idx2d.shape[0] * idx2d.shape[1]
    n_blocks = n_chunks_per_w // _GPB
    block_rows = _GPB * _CHUNK
    mesh = plsc.VectorSubcoreMesh(core_axis_name="c", subcore_axis_name="s")

    @functools.partial(
        pl.kernel,
        out_type=jax.ShapeDtypeStruct((n_idx, dim), jnp.float32),
        mesh=mesh,
        scratch_types=[
            pltpu.VMEM((n_chunks_per_w, _CHUNK), jnp.int32),
            pltpu.VMEM((_NBUF, block_rows, dim), jnp.float32),
        ]
        + [pltpu.SemaphoreType.DMA] * _NBUF,
        compiler_params=pltpu.CompilerParams(use_tc_tiling_on_sc=False),
    )
    def k(idx_hbm, table_hbm, out_hbm, idx_v, rows_v, *sems):
        wid = lax.axis_index("s") * _NC + lax.axis_index("c")
        row0 = wid * n_chunks_per_w
        pltpu.sync_copy(idx_hbm.at[pl.ds(row0, n_chunks_per_w)], idx_v)

        def fire(buf, blk):
            buf_ref = rows_v.at[buf]
            for g in range(_GPB):
                pltpu.async_copy(
                    table_hbm.at[idx_v.at[blk * _GPB + g]],
                    buf_ref.at[pl.ds(g * _CHUNK, _CHUNK)],
                    sems[buf],
                )

        def drain(buf):
            buf_ref = rows_v.at[buf]
            for g in range(_GPB):
                pltpu.make_async_copy(
                    table_hbm.at[idx_v.at[g]],
                    buf_ref.at[pl.ds(g * _CHUNK, _CHUNK)],
                    sems[buf],
                ).wait()

        for b in range(_NBUF):
            fire(b, b)

        @pl.loop(0, n_blocks, step=_NBUF)
        def block_body(blk0):
            for b in range(_NBUF):
                blk = blk0 + b
                drain(b)
                pltpu.sync_copy(
                    rows_v.at[b],
                    out_hbm.at[pl.ds((row0 + blk * _GPB) * _CHUNK, block_rows)],
                )

                @pl.when(blk + _NBUF < n_blocks)
                def _():
                    fire(b, blk + _NBUF)

    return k(idx2d, table)


def kernel(indices, embeddings):
    b, s = indices.shape
    dim = embeddings.shape[1]
    n_idx = b * s
    assert n_idx % (_CHUNK * _GPB * _NW) == 0
    idx2d = indices.reshape(n_idx // _CHUNK, _CHUNK)
    out = _gather(idx2d, embeddings, n_chunks_per_w=n_idx // (_CHUNK * _NW), dim=dim)
    return out.reshape(b, s, dim)
